# Initial kernel scaffold; baseline (speedup 1.0000x reference)
import jax
import jax.numpy as jnp
from jax.experimental import pallas as pl


def kernel(x, Wq, Wk, Wv, Wo):
    raise NotImplementedError("write your pallas kernel here")


# speedup vs baseline: 10.5211x; 10.5211x over previous
import functools
import math

import jax
import jax.numpy as jnp
import numpy as np
from jax.experimental import pallas as pl
from jax.experimental.pallas import tpu as pltpu

B, L, C = 2, 4096, 1024
H = 16
Dh = C // H
K = 64
QB = 128


@functools.cache
def _neighbor_counts() -> np.ndarray:
    with jax.ensure_compile_time_eval():
        rand_idx = np.asarray(
            jax.random.randint(jax.random.key(42), (L, K - 1), 0, L,
                               dtype=jnp.int32))
    self_idx = np.arange(L, dtype=np.int32).reshape(L, 1)
    idx = np.concatenate([self_idx, rand_idx], axis=-1)
    cnt = np.zeros((L, L), dtype=np.int8)
    np.add.at(cnt, (np.repeat(np.arange(L), K), idx.reshape(-1)), 1)
    return cnt


def _mm_kernel(x_ref, w_ref, o_ref):
    o_ref[...] = jnp.dot(x_ref[...], w_ref[...],
                         preferred_element_type=jnp.float32)


def _matmul(x, w, bm, bn):
    m, k = x.shape
    _, n = w.shape
    return pl.pallas_call(
        _mm_kernel,
        grid=(m // bm, n // bn),
        in_specs=[
            pl.BlockSpec((bm, k), lambda i, j: (i, 0)),
            pl.BlockSpec((k, bn), lambda i, j: (0, j)),
        ],
        out_specs=pl.BlockSpec((bm, bn), lambda i, j: (i, j)),
        out_shape=jax.ShapeDtypeStruct((m, n), jnp.float32),
    )(x, w)


def _attn_kernel(q_ref, k_ref, v_ref, c_ref, o_ref):
    i = pl.program_id(1)
    q = q_ref[0, 0, :, :] * (1.0 / math.sqrt(Dh))
    k = k_ref[0, 0, :, :]
    v = v_ref[0, 0, :, :]
    cnt = c_ref[pl.ds(i * QB, QB), :].astype(jnp.float32)
    s = jax.lax.dot_general(q, k, (((1,), (1,)), ((), ())),
                            preferred_element_type=jnp.float32)
    s = jnp.where(cnt > 0.0, s, -1e30)
    m = jnp.max(s, axis=1, keepdims=True)
    p = cnt * jnp.exp(s - m)
    denom = jnp.sum(p, axis=1, keepdims=True)
    o = jnp.dot(p, v, preferred_element_type=jnp.float32)
    o_ref[0, 0, :, :] = o / denom


def _attention(q, k, v, cnt):
    return pl.pallas_call(
        _attn_kernel,
        grid=(B * H, L // QB),
        in_specs=[
            pl.BlockSpec((1, 1, QB, Dh), lambda bh, i: (bh // H, bh % H, i, 0)),
            pl.BlockSpec((1, 1, L, Dh), lambda bh, i: (bh // H, bh % H, 0, 0)),
            pl.BlockSpec((1, 1, L, Dh), lambda bh, i: (bh // H, bh % H, 0, 0)),
            pl.BlockSpec((L, L), lambda bh, i: (0, 0)),
        ],
        out_specs=pl.BlockSpec((1, 1, QB, Dh),
                               lambda bh, i: (bh // H, bh % H, i, 0)),
        out_shape=jax.ShapeDtypeStruct((B, H, L, Dh), jnp.float32),
        compiler_params=pltpu.CompilerParams(
            dimension_semantics=("arbitrary", "arbitrary"),
        ),
    )(q, k, v, cnt)


def kernel(x, Wq, Wk, Wv, Wo):
    cnt = jnp.asarray(_neighbor_counts())
    w_qkv = jnp.concatenate([Wq.T, Wk.T, Wv.T], axis=1)
    qkv = _matmul(x.reshape(B * L, C), w_qkv, bm=512, bn=512)
    qkv = qkv.reshape(B, L, 3, H, Dh).transpose(2, 0, 3, 1, 4)
    q, k, v = qkv[0], qkv[1], qkv[2]
    attn = _attention(q, k, v, cnt)
    attn = attn.transpose(0, 2, 1, 3).reshape(B * L, C)
    out = _matmul(attn, Wo.T, bm=512, bn=512)
    return out.reshape(B, L, C)
